# Initial kernel scaffold; baseline (speedup 1.0000x reference)
import jax
import jax.numpy as jnp
from jax.experimental import pallas as pl


def kernel(x, W1, b1, gamma, beta, W2, b2):
    raise NotImplementedError("write your pallas kernel here")


# speedup vs baseline: 33.9941x; 33.9941x over previous
import functools

import numpy as np
import jax
import jax.numpy as jnp
from jax import lax
from jax.experimental import pallas as pl
from jax.experimental.pallas import tpu as pltpu
from jax.experimental.pallas import tpu_sc as plsc

NBINS = 256
LANES = 16
NCORES = 2
NSUB = 16
NWORK = NCORES * NSUB


def _dct_matrix(n):
    i = np.arange(n)
    k = np.arange(n)[:, None]
    m = np.cos(np.pi * (2 * i + 1) * k / (2 * n))
    m[0] *= np.sqrt(1.0 / n)
    m[1:] *= np.sqrt(2.0 / n)
    return m.astype(np.float32)


def _dct_bin_body(x_ref, d_ref, dt_ref, idx_ref):
    xb = x_ref[0]
    tmp = jnp.dot(d_ref[...], xb, preferred_element_type=jnp.float32)
    coeffs = jnp.dot(tmp, dt_ref[...], preferred_element_type=jnp.float32)
    mag = jnp.abs(coeffs)
    m = jnp.max(mag)
    scaled = jnp.where(m > 0, mag / m, jnp.zeros_like(mag))
    q = jnp.clip(jnp.floor(scaled * NBINS).astype(jnp.int32), 0, NBINS - 1)
    idx_ref[0] = q


def _dct_bin(flat, d, dt):
    p, h, w = flat.shape
    return pl.pallas_call(
        _dct_bin_body,
        grid=(p,),
        in_specs=[
            pl.BlockSpec((1, h, w), lambda i: (i, 0, 0)),
            pl.BlockSpec((h, h), lambda i: (0, 0)),
            pl.BlockSpec((w, w), lambda i: (0, 0)),
        ],
        out_specs=pl.BlockSpec((1, h, w), lambda i: (i, 0, 0)),
        out_shape=jax.ShapeDtypeStruct((p, h, w), jnp.int32),
    )(flat, d, dt)


def _make_hist(nplanes, plane_elems):
    ppw = nplanes // NWORK
    nvec = plane_elems // LANES
    mesh = plsc.VectorSubcoreMesh(
        core_axis_name="c", subcore_axis_name="s",
        num_cores=NCORES, num_subcores=NSUB)

    @functools.partial(
        pl.kernel,
        out_type=jax.ShapeDtypeStruct((nplanes, NBINS), jnp.float32),
        mesh=mesh,
        scratch_types=[
            pltpu.VMEM((plane_elems,), jnp.int32),
            pltpu.VMEM((NBINS * LANES,), jnp.float32),
            pltpu.VMEM((NBINS,), jnp.float32),
        ],
        compiler_params=pltpu.CompilerParams(needs_layout_passes=False),
    )
    def hist_kernel(idx_hbm, out_hbm, idx_v, tab_v, row_v):
        wid = lax.axis_index("s") * NCORES + lax.axis_index("c")
        lane_base = lax.iota(jnp.int32, LANES) * NBINS
        ones = jnp.ones((LANES,), jnp.float32)
        zeros = jnp.zeros((LANES,), jnp.float32)

        for pp in range(ppw):
            p = wid * ppw + pp
            pltpu.sync_copy(idx_hbm.at[p], idx_v)

            def zero_body(i, _):
                tab_v[pl.ds(i * LANES, LANES)] = zeros
                return 0

            lax.fori_loop(0, NBINS, zero_body, 0)

            def scat_body(i, _):
                q = idx_v[pl.ds(i * LANES, LANES)]
                plsc.addupdate_scatter(tab_v, [q + lane_base], ones)
                return 0

            lax.fori_loop(0, nvec, scat_body, 0)

            for j in range(NBINS // LANES):
                acc = tab_v[pl.ds(j * LANES, LANES)]
                for l in range(1, LANES):
                    acc = acc + tab_v[pl.ds(l * NBINS + j * LANES, LANES)]
                row_v[pl.ds(j * LANES, LANES)] = acc

            pltpu.sync_copy(row_v, out_hbm.at[p])

    return hist_kernel


def _mlp_body(hist_ref, w1_ref, b1_ref, g_ref, be_ref, w2_ref, b2_ref, out_ref,
              *, inv_n):
    h = jnp.dot(hist_ref[...], w1_ref[...], preferred_element_type=jnp.float32)
    h = h * inv_n + b1_ref[...]
    mu = jnp.mean(h, axis=-1, keepdims=True)
    var = jnp.mean((h - mu) * (h - mu), axis=-1, keepdims=True)
    hn = (h - mu) / jnp.sqrt(var + 1e-5) * g_ref[...] + be_ref[...]
    hr = jnp.maximum(hn, 0.0)
    out_ref[...] = jnp.dot(hr, w2_ref[...], preferred_element_type=jnp.float32) + b2_ref[...]


def _mlp(counts, w1, b1, g, be, w2, b2, inv_n):
    p = counts.shape[0]
    fdim = w2.shape[1]
    return pl.pallas_call(
        functools.partial(_mlp_body, inv_n=inv_n),
        out_shape=jax.ShapeDtypeStruct((p, fdim), jnp.float32),
    )(counts, w1, b1.reshape(1, -1), g.reshape(1, -1), be.reshape(1, -1),
      w2, b2.reshape(1, -1))


def kernel(x, W1, b1, gamma, beta, W2, b2):
    B, C, H, W = x.shape
    P = B * C
    flat = x.reshape(P, H, W)
    d = jnp.asarray(_dct_matrix(H))
    idx = _dct_bin(flat, d, d.T)
    counts = _make_hist(P, H * W)(idx.reshape(P, H * W))
    return _mlp(counts, W1, b1, gamma, beta, W2, b2, 1.0 / (H * W))


# speedup vs baseline: 92.8318x; 2.7308x over previous
import functools

import numpy as np
import jax
import jax.numpy as jnp
from jax import lax
from jax.experimental import pallas as pl
from jax.experimental.pallas import tpu as pltpu
from jax.experimental.pallas import tpu_sc as plsc

NBINS = 256
LANES = 16
NCORES = 2
NSUB = 16
NWORK = NCORES * NSUB
PLANES_PER_STEP = 4


def _dct_matrix(n):
    i = np.arange(n)
    k = np.arange(n)[:, None]
    m = np.cos(np.pi * (2 * i + 1) * k / (2 * n))
    m[0] *= np.sqrt(1.0 / n)
    m[1:] *= np.sqrt(2.0 / n)
    return m.astype(np.float32)


def _dct_bin_body(x_ref, d_ref, dt_ref, idx_ref):
    for i in range(PLANES_PER_STEP):
        xb = x_ref[i]
        tmp = jnp.dot(d_ref[...], xb, preferred_element_type=jnp.float32)
        coeffs = jnp.dot(tmp, dt_ref[...], preferred_element_type=jnp.float32)
        mag = jnp.abs(coeffs)
        m = jnp.max(mag)
        r = jnp.where(m > 0, NBINS / m, 0.0)
        q = jnp.minimum(jnp.floor(mag * r), NBINS - 1.0).astype(jnp.int32)
        hq = q.shape[0] // 4
        w = (q[0 * hq:1 * hq] | (q[1 * hq:2 * hq] << 8)
             | (q[2 * hq:3 * hq] << 16) | (q[3 * hq:4 * hq] << 24))
        idx_ref[i] = w


def _dct_bin(flat, d, dt):
    p, h, w = flat.shape
    pb = PLANES_PER_STEP
    return pl.pallas_call(
        _dct_bin_body,
        grid=(p // pb,),
        in_specs=[
            pl.BlockSpec((pb, h, w), lambda i: (i, 0, 0)),
            pl.BlockSpec((h, h), lambda i: (0, 0)),
            pl.BlockSpec((w, w), lambda i: (0, 0)),
        ],
        out_specs=pl.BlockSpec((pb, h // 4, w), lambda i: (i, 0, 0)),
        out_shape=jax.ShapeDtypeStruct((p, h // 4, w), jnp.int32),
    )(flat, d, dt)


def _make_hist(nplanes, nrows, ncols):
    ppw = nplanes // NWORK
    vec_per_row = ncols // LANES
    mesh = plsc.VectorSubcoreMesh(
        core_axis_name="c", subcore_axis_name="s",
        num_cores=NCORES, num_subcores=NSUB)

    @functools.partial(
        pl.kernel,
        out_type=jax.ShapeDtypeStruct((nplanes, NBINS), jnp.float32),
        mesh=mesh,
        scratch_types=[
            pltpu.VMEM((nrows, ncols), jnp.int32),
            pltpu.VMEM((nrows, ncols), jnp.int32),
            pltpu.VMEM((NBINS * LANES,), jnp.float32),
            pltpu.VMEM((NBINS,), jnp.float32),
            pltpu.SemaphoreType.DMA,
            pltpu.SemaphoreType.DMA,
        ],
        compiler_params=pltpu.CompilerParams(needs_layout_passes=False),
    )
    def hist_kernel(idx_hbm, out_hbm, buf0, buf1, tab_v, row_v, sem0, sem1):
        wid = lax.axis_index("s") * NCORES + lax.axis_index("c")
        lane_base = lax.iota(jnp.int32, LANES) * NBINS
        ones = jnp.ones((LANES,), jnp.float32)
        zeros = jnp.zeros((LANES,), jnp.float32)
        bufs = (buf0, buf1)
        sems = (sem0, sem1)

        p0 = wid * ppw
        copies = [pltpu.async_copy(idx_hbm.at[p0], buf0, sem0)]

        for pp in range(ppw):
            buf = bufs[pp % 2]
            if pp + 1 < ppw:
                copies.append(pltpu.async_copy(
                    idx_hbm.at[p0 + pp + 1], bufs[(pp + 1) % 2],
                    sems[(pp + 1) % 2]))

            def zero_body(z, _):
                for u in range(16):
                    tab_v[pl.ds((z * 16 + u) * LANES, LANES)] = zeros
                return 0

            lax.fori_loop(0, NBINS // 16, zero_body, 0)

            copies[pp].wait()

            def scat_body(i, _):
                for u in range(vec_per_row):
                    w = buf[i, pl.ds(u * LANES, LANES)]
                    q0 = w & 255
                    q1 = (w >> 8) & 255
                    q2 = (w >> 16) & 255
                    q3 = (w >> 24) & 255
                    for q in (q0, q1, q2, q3):
                        plsc.addupdate_scatter(tab_v, [q + lane_base], ones)
                return 0

            lax.fori_loop(0, nrows, scat_body, 0)

            def red_body(j, _):
                acc = tab_v[pl.ds(j * LANES, LANES)]
                for l in range(1, LANES):
                    acc = acc + tab_v[pl.ds(l * NBINS + j * LANES, LANES)]
                row_v[pl.ds(j * LANES, LANES)] = acc
                return 0

            lax.fori_loop(0, NBINS // LANES, red_body, 0)
            pltpu.sync_copy(row_v, out_hbm.at[p0 + pp])

    return hist_kernel


def _mlp_body(hist_ref, w1_ref, b1_ref, g_ref, be_ref, w2_ref, b2_ref, out_ref,
              *, inv_n):
    h = jnp.dot(hist_ref[...], w1_ref[...], preferred_element_type=jnp.float32)
    h = h * inv_n + b1_ref[...]
    mu = jnp.mean(h, axis=-1, keepdims=True)
    var = jnp.mean((h - mu) * (h - mu), axis=-1, keepdims=True)
    hn = (h - mu) / jnp.sqrt(var + 1e-5) * g_ref[...] + be_ref[...]
    hr = jnp.maximum(hn, 0.0)
    out_ref[...] = jnp.dot(hr, w2_ref[...], preferred_element_type=jnp.float32) + b2_ref[...]


def _mlp(counts, w1, b1, g, be, w2, b2, inv_n):
    p = counts.shape[0]
    fdim = w2.shape[1]
    return pl.pallas_call(
        functools.partial(_mlp_body, inv_n=inv_n),
        out_shape=jax.ShapeDtypeStruct((p, fdim), jnp.float32),
    )(counts, w1, b1.reshape(1, -1), g.reshape(1, -1), be.reshape(1, -1),
      w2, b2.reshape(1, -1))


def kernel(x, W1, b1, gamma, beta, W2, b2):
    B, C, H, W = x.shape
    P = B * C
    flat = x.reshape(P, H, W)
    d = jnp.asarray(_dct_matrix(H))
    idx = _dct_bin(flat, d, d.T)
    counts = _make_hist(P, H // 4, W)(idx)
    return _mlp(counts, W1, b1, gamma, beta, W2, b2, 1.0 / (H * W))


# speedup vs baseline: 146.1482x; 1.5743x over previous
import functools

import numpy as np
import jax
import jax.numpy as jnp
from jax import lax
from jax.experimental import pallas as pl
from jax.experimental.pallas import tpu as pltpu
from jax.experimental.pallas import tpu_sc as plsc

NBINS = 256
LANES = 16
NCORES = 2
NSUB = 16
NWORK = NCORES * NSUB
PLANES_PER_STEP = 16


def _dct_matrix(n):
    i = np.arange(n)
    k = np.arange(n)[:, None]
    m = np.cos(np.pi * (2 * i + 1) * k / (2 * n))
    m[0] *= np.sqrt(1.0 / n)
    m[1:] *= np.sqrt(2.0 / n)
    return m.astype(np.float32)


def _dct_bin_body(x_ref, d_ref, dt_ref, idx_ref):
    for i in range(PLANES_PER_STEP):
        xb = x_ref[i]
        tmp = jnp.dot(d_ref[...], xb, preferred_element_type=jnp.float32)
        coeffs = jnp.dot(tmp, dt_ref[...], preferred_element_type=jnp.float32)
        mag = jnp.abs(coeffs)
        m = jnp.max(mag)
        r = jnp.where(m > 0, NBINS / m, 0.0)
        q = jnp.minimum(jnp.floor(mag * r), NBINS - 1.0).astype(jnp.int32)
        hq = q.shape[0] // 4
        w = (q[0 * hq:1 * hq] | (q[1 * hq:2 * hq] << 8)
             | (q[2 * hq:3 * hq] << 16) | (q[3 * hq:4 * hq] << 24))
        idx_ref[i] = w


def _dct_bin(flat, d, dt):
    p, h, w = flat.shape
    pb = PLANES_PER_STEP
    return pl.pallas_call(
        _dct_bin_body,
        grid=(p // pb,),
        in_specs=[
            pl.BlockSpec((pb, h, w), lambda i: (i, 0, 0)),
            pl.BlockSpec((h, h), lambda i: (0, 0)),
            pl.BlockSpec((w, w), lambda i: (0, 0)),
        ],
        out_specs=pl.BlockSpec((pb, h // 4, w), lambda i: (i, 0, 0)),
        out_shape=jax.ShapeDtypeStruct((p, h // 4, w), jnp.int32),
    )(flat, d, dt)


def _make_hist(nplanes, nrows, ncols):
    ppw = nplanes // NWORK
    vec_per_row = ncols // LANES
    mesh = plsc.VectorSubcoreMesh(
        core_axis_name="c", subcore_axis_name="s",
        num_cores=NCORES, num_subcores=NSUB)

    @functools.partial(
        pl.kernel,
        out_type=jax.ShapeDtypeStruct((nplanes, NBINS), jnp.float32),
        mesh=mesh,
        scratch_types=[
            pltpu.VMEM((nrows, ncols), jnp.int32),
            pltpu.VMEM((nrows, ncols), jnp.int32),
            pltpu.VMEM((NBINS * LANES,), jnp.float32),
            pltpu.VMEM((NBINS,), jnp.float32),
            pltpu.SemaphoreType.DMA,
            pltpu.SemaphoreType.DMA,
        ],
        compiler_params=pltpu.CompilerParams(needs_layout_passes=False),
    )
    def hist_kernel(idx_hbm, out_hbm, buf0, buf1, tab_v, row_v, sem0, sem1):
        wid = lax.axis_index("s") * NCORES + lax.axis_index("c")
        lane_base = lax.iota(jnp.int32, LANES) * NBINS
        ones = jnp.ones((LANES,), jnp.float32)
        zeros = jnp.zeros((LANES,), jnp.float32)
        bufs = (buf0, buf1)
        sems = (sem0, sem1)

        p0 = wid * ppw
        copies = [pltpu.async_copy(idx_hbm.at[p0], buf0, sem0)]

        for pp in range(ppw):
            buf = bufs[pp % 2]
            if pp + 1 < ppw:
                copies.append(pltpu.async_copy(
                    idx_hbm.at[p0 + pp + 1], bufs[(pp + 1) % 2],
                    sems[(pp + 1) % 2]))

            @plsc.parallel_loop(0, NBINS // 16)
            def _(z):
                for u in range(16):
                    tab_v[pl.ds((z * 16 + u) * LANES, LANES)] = zeros

            copies[pp].wait()

            @plsc.parallel_loop(0, nrows, unroll=2)
            def _(i):
                for u in range(vec_per_row):
                    w = buf[i, pl.ds(u * LANES, LANES)]
                    q0 = w & 255
                    q1 = (w >> 8) & 255
                    q2 = (w >> 16) & 255
                    q3 = (w >> 24) & 255
                    for q in (q0, q1, q2, q3):
                        plsc.addupdate_scatter(tab_v, [q + lane_base], ones)

            @plsc.parallel_loop(0, NBINS // LANES)
            def _(j):
                acc = tab_v[pl.ds(j * LANES, LANES)]
                for l in range(1, LANES):
                    acc = acc + tab_v[pl.ds(l * NBINS + j * LANES, LANES)]
                row_v[pl.ds(j * LANES, LANES)] = acc
            pltpu.sync_copy(row_v, out_hbm.at[p0 + pp])

    return hist_kernel


def _mlp_body(hist_ref, w1_ref, b1_ref, g_ref, be_ref, w2_ref, b2_ref, out_ref,
              *, inv_n):
    h = jnp.dot(hist_ref[...], w1_ref[...], preferred_element_type=jnp.float32)
    h = h * inv_n + b1_ref[...]
    mu = jnp.mean(h, axis=-1, keepdims=True)
    var = jnp.mean((h - mu) * (h - mu), axis=-1, keepdims=True)
    hn = (h - mu) / jnp.sqrt(var + 1e-5) * g_ref[...] + be_ref[...]
    hr = jnp.maximum(hn, 0.0)
    out_ref[...] = jnp.dot(hr, w2_ref[...], preferred_element_type=jnp.float32) + b2_ref[...]


def _mlp(counts, w1, b1, g, be, w2, b2, inv_n):
    p = counts.shape[0]
    fdim = w2.shape[1]
    return pl.pallas_call(
        functools.partial(_mlp_body, inv_n=inv_n),
        out_shape=jax.ShapeDtypeStruct((p, fdim), jnp.float32),
    )(counts, w1, b1.reshape(1, -1), g.reshape(1, -1), be.reshape(1, -1),
      w2, b2.reshape(1, -1))


def kernel(x, W1, b1, gamma, beta, W2, b2):
    B, C, H, W = x.shape
    P = B * C
    flat = x.reshape(P, H, W)
    d = jnp.asarray(_dct_matrix(H))
    idx = _dct_bin(flat, d, d.T)
    counts = _make_hist(P, H // 4, W)(idx)
    return _mlp(counts, W1, b1, gamma, beta, W2, b2, 1.0 / (H * W))
